# Initial kernel scaffold; baseline (speedup 1.0000x reference)
import jax
import jax.numpy as jnp
from jax.experimental import pallas as pl


def kernel(features, edge_index, W1, b1, g1, be1, W2, b2, g2, be2):
    raise NotImplementedError("write your pallas kernel here")


# speedup vs baseline: 4.9750x; 4.9750x over previous
import functools

import jax
import jax.numpy as jnp
from jax import lax
from jax.experimental import pallas as pl
from jax.experimental.pallas import tpu as pltpu
from jax.experimental.pallas import tpu_sc as plsc

N = 10000
E = 320000
D = 128

NC = 2
NS = 16
NW = NC * NS

CHUNK = 128
EPW = E // NW
NCHUNK = -(-EPW // CHUNK)
EPW_PAD = NCHUNK * CHUNK
E_PAD = EPW_PAD * NW

ROWS_PER_TILE = -(-N // (NS * CHUNK)) * CHUNK
N_PAD = ROWS_PER_TILE * NS


def _sc_aggregate_body(x_hbm, src_hbm, dst_hbm, zeros_hbm, ones_hbm,
                       agg_out, deg_out,
                       idx_src, idx_dst, rows, ones_v, acc, dacc, sem):
  c = lax.axis_index("c")
  s = lax.axis_index("s")
  wid = c * NS + s

  pltpu.sync_copy(zeros_hbm, rows)
  base = s * ROWS_PER_TILE
  for i in range(ROWS_PER_TILE // CHUNK):
    pltpu.sync_copy(rows, acc.at[pl.ds(base + i * CHUNK, CHUNK)])
    pltpu.sync_copy(rows.at[0], dacc.at[pl.ds(base + i * CHUNK, CHUNK)])
  pltpu.sync_copy(ones_hbm, ones_v)
  pltpu.sync_copy(src_hbm.at[wid], idx_src)
  pltpu.sync_copy(dst_hbm.at[wid], idx_dst)
  plsc.subcore_barrier()

  def chunk_step(i, carry):
    pltpu.async_copy(x_hbm.at[idx_src.at[i]], rows, sem).wait()
    pltpu.sync_copy(rows, acc.at[idx_dst.at[i]], add=True)
    pltpu.sync_copy(ones_v, dacc.at[idx_dst.at[i]], add=True)
    return carry

  lax.fori_loop(0, NCHUNK, chunk_step, 0)

  plsc.subcore_barrier()
  pltpu.sync_copy(acc.at[pl.ds(base, ROWS_PER_TILE)],
                  agg_out.at[c, pl.ds(base, ROWS_PER_TILE)])
  pltpu.sync_copy(dacc.at[pl.ds(base, ROWS_PER_TILE)],
                  deg_out.at[c, pl.ds(base, ROWS_PER_TILE)])


_sc_aggregate = functools.partial(
    pl.kernel,
    out_type=(
        jax.ShapeDtypeStruct((NC, N_PAD, D), jnp.float32),
        jax.ShapeDtypeStruct((NC, N_PAD), jnp.float32),
    ),
    mesh=plsc.VectorSubcoreMesh(core_axis_name="c", subcore_axis_name="s"),
    scratch_types=[
        pltpu.VMEM((NCHUNK, CHUNK), jnp.int32),
        pltpu.VMEM((NCHUNK, CHUNK), jnp.int32),
        pltpu.VMEM((CHUNK, D), jnp.float32),
        pltpu.VMEM((CHUNK,), jnp.float32),
        pltpu.VMEM_SHARED((N_PAD, D), jnp.float32),
        pltpu.VMEM_SHARED((N_PAD,), jnp.float32),
        pltpu.SemaphoreType.DMA,
    ],
)(_sc_aggregate_body)


def _tc_dense_body(x_ref, a0, a1, d0, d1, w, b, g, be, o):
  agg = a0[...] + a1[...]
  deg = jnp.maximum(d0[...] + d1[...], 1.0)
  agg = agg / deg
  h = jnp.dot(x_ref[...], w[:D, :], preferred_element_type=jnp.float32)
  h = h + jnp.dot(agg, w[D:, :], preferred_element_type=jnp.float32)
  h = jnp.maximum(h + b[...], 0.0)
  mean = jnp.mean(h, axis=0, keepdims=True)
  zm = h - mean
  var = jnp.mean(zm * zm, axis=0, keepdims=True)
  hn = zm * lax.rsqrt(var + 1e-5) * g[...] + be[...]
  nrm = jnp.sqrt(jnp.sum(hn * hn, axis=1, keepdims=True))
  o[...] = hn / (nrm + 1e-6)


def _tc_dense(x, aggp, degp, w, b, g, be):
  return pl.pallas_call(
      _tc_dense_body,
      out_shape=jax.ShapeDtypeStruct((N, D), jnp.float32),
  )(x, aggp[0, :N], aggp[1, :N],
    degp[0, :N].reshape(N, 1), degp[1, :N].reshape(N, 1),
    w, b.reshape(1, D), g.reshape(1, D), be.reshape(1, D))


def kernel(features, edge_index, W1, b1, g1, be1, W2, b2, g2, be2):
  src = edge_index[0]
  dst = edge_index[1]
  pad = E_PAD - E
  src_p = jnp.concatenate([src, jnp.zeros((pad,), jnp.int32)]).reshape(NW, NCHUNK, CHUNK)
  dst_p = jnp.concatenate([dst, jnp.full((pad,), N, jnp.int32)]).reshape(NW, NCHUNK, CHUNK)
  zeros = jnp.zeros((CHUNK, D), jnp.float32)
  ones = jnp.ones((CHUNK,), jnp.float32)

  agg1, deg = _sc_aggregate(features, src_p, dst_p, zeros, ones)
  h1 = _tc_dense(features, agg1, deg, W1, b1, g1, be1)
  agg2, _ = _sc_aggregate(h1, src_p, dst_p, zeros, ones)
  h2 = _tc_dense(h1, agg2, deg, W2, b2, g2, be2)
  return h2
